# baseline (speedup 1.0000x reference)
import functools

import jax
import jax.numpy as jnp
from jax import lax
from jax.experimental import pallas as pl
from jax.experimental.pallas import tpu as pltpu
from jax.experimental.pallas import tpu_sc as plsc

_SC_CH = 16384
_LP = (3.51102136e-06, 0.999792362, -0.496977431, 0.314589174,
       -0.188780824, 0.0817256453, -0.0172077992)


def _log1p_poly(e):
    acc = jnp.full_like(e, _LP[6])
    for coef in _LP[5::-1]:
        acc = acc * e + coef
    return acc


def _masked_bce(x, t):
    tf = t.astype(jnp.float32)
    zf = jnp.maximum(tf, 0.0)
    mf = jnp.minimum(tf + 1.0, 1.0)
    e = jnp.exp(-jnp.abs(x))
    sp = jnp.maximum(x, 0.0) + _log1p_poly(e)
    return mf * sp - x * zf, mf


def _make_sc_loss(n):
    info = plsc.get_sparse_core_info()
    nc, ns = info.num_cores, info.num_subcores
    nw = nc * ns
    per_w = n // nw
    n_ch = per_w // _SC_CH

    mesh = plsc.VectorSubcoreMesh(core_axis_name="c", subcore_axis_name="s")

    @functools.partial(
        pl.kernel, mesh=mesh,
        out_type=jax.ShapeDtypeStruct((2, nw, 16), jnp.float32),
        scratch_types=[
            pltpu.VMEM((_SC_CH,), jnp.float32),
            pltpu.VMEM((_SC_CH,), jnp.int32),
            pltpu.VMEM((2, 16), jnp.float32),
        ],
    )
    def sc_loss(x_hbm, t_hbm, out_hbm, x_v, t_v, part_v):
        wid = lax.axis_index("s") * nc + lax.axis_index("c")
        base = wid * per_w

        def chunk_body(ci, carry):
            pltpu.sync_copy(x_hbm.at[pl.ds(base + ci * _SC_CH, _SC_CH)], x_v)
            pltpu.sync_copy(t_hbm.at[pl.ds(base + ci * _SC_CH, _SC_CH)], t_v)

            def vec_body(j, sc2):
                s2, c2 = sc2
                x = x_v[pl.ds(j * 16, 16)]
                t = t_v[pl.ds(j * 16, 16)]
                ds, dc = _masked_bce(x, t)
                return s2 + ds, c2 + dc

            return lax.fori_loop(0, _SC_CH // 16, vec_body, carry, unroll=8)

        z = jnp.zeros((16,), jnp.float32)
        s, c = lax.fori_loop(0, n_ch, chunk_body, (z, z))
        part_v[0] = s
        part_v[1] = c
        pltpu.sync_copy(part_v.at[0], out_hbm.at[0, wid])
        pltpu.sync_copy(part_v.at[1], out_hbm.at[1, wid])

    return sc_loss


def kernel(output, target):
    n = output.shape[0]
    parts = _make_sc_loss(n)(output, target)
    return jnp.sum(parts[0]) / jnp.sum(parts[1])


# speedup vs baseline: 2.4818x; 2.4818x over previous
import functools

import jax
import jax.numpy as jnp
from jax import lax
from jax.experimental import pallas as pl
from jax.experimental.pallas import tpu as pltpu
from jax.experimental.pallas import tpu_sc as plsc

_SC_CH = 16384
_SC_UNIT = 524288
_SC_UNITS = 4

_TC_CHUNK = 524288
_TC_SUB = 16384
_TC_ROWS = _TC_SUB // 128

_LP = (3.51102136e-06, 0.999792362, -0.496977431, 0.314589174,
       -0.188780824, 0.0817256453, -0.0172077992)


def _log1p_poly(e):
    acc = jnp.full_like(e, _LP[6])
    for coef in _LP[5::-1]:
        acc = acc * e + coef
    return acc


def _masked_bce(x, t, log1p_fn):
    tf = t.astype(jnp.float32)
    zf = jnp.maximum(tf, 0.0)
    mf = jnp.minimum(tf + 1.0, 1.0)
    e = jnp.exp(-jnp.abs(x))
    sp = jnp.maximum(x, 0.0) + log1p_fn(e)
    return mf * sp - x * zf, mf


def _make_sc_loss(n_skip, n_sc):
    info = plsc.get_sparse_core_info()
    nc, ns = info.num_cores, info.num_subcores
    nw = nc * ns
    per_w = n_sc // nw
    n_ch = per_w // _SC_CH

    mesh = plsc.VectorSubcoreMesh(core_axis_name="c", subcore_axis_name="s")

    @functools.partial(
        pl.kernel, mesh=mesh,
        out_type=jax.ShapeDtypeStruct((2, nw, 16), jnp.float32),
        scratch_types=[
            pltpu.VMEM((_SC_CH,), jnp.float32),
            pltpu.VMEM((_SC_CH,), jnp.int32),
            pltpu.VMEM((2, 16), jnp.float32),
        ],
    )
    def sc_loss(x_hbm, t_hbm, out_hbm, x_v, t_v, part_v):
        wid = lax.axis_index("s") * nc + lax.axis_index("c")
        base = n_skip + wid * per_w

        def chunk_body(ci, carry):
            pltpu.sync_copy(x_hbm.at[pl.ds(base + ci * _SC_CH, _SC_CH)], x_v)
            pltpu.sync_copy(t_hbm.at[pl.ds(base + ci * _SC_CH, _SC_CH)], t_v)

            def vec_body(j, sc2):
                s2, c2 = sc2
                x = x_v[pl.ds(j * 16, 16)]
                t = t_v[pl.ds(j * 16, 16)]
                ds, dc = _masked_bce(x, t, _log1p_poly)
                return s2 + ds, c2 + dc

            return lax.fori_loop(0, _SC_CH // 16, vec_body, carry, unroll=8)

        z = jnp.zeros((16,), jnp.float32)
        s, c = lax.fori_loop(0, n_ch, chunk_body, (z, z))
        part_v[0] = s
        part_v[1] = c
        pltpu.sync_copy(part_v.at[0], out_hbm.at[0, wid])
        pltpu.sync_copy(part_v.at[1], out_hbm.at[1, wid])

    return sc_loss


def _tc_body(x_ref, t_ref, out_ref, acc_ref):
    i = pl.program_id(0)

    @pl.when(i == 0)
    def _init():
        acc_ref[...] = jnp.zeros_like(acc_ref)

    def step(j, carry):
        s, c = carry
        x = x_ref[pl.ds(j * _TC_SUB, _TC_SUB)].reshape(_TC_ROWS, 128)
        t = t_ref[pl.ds(j * _TC_SUB, _TC_SUB)].reshape(_TC_ROWS, 128)
        ds, dc = _masked_bce(x, t, jnp.log1p)
        return s + ds, c + dc

    init = (jnp.zeros((_TC_ROWS, 128), jnp.float32),
            jnp.zeros((_TC_ROWS, 128), jnp.float32))
    s, c = jax.lax.fori_loop(0, _TC_CHUNK // _TC_SUB, step, init, unroll=2)
    acc_ref[0] += s
    acc_ref[1] += c

    @pl.when(i == pl.num_programs(0) - 1)
    def _fin():
        out_ref[0] = jnp.sum(acc_ref[0])
        out_ref[1] = jnp.sum(acc_ref[1])


def _tc_loss(output, target, n_tc):
    grid = n_tc // _TC_CHUNK
    return pl.pallas_call(
        _tc_body,
        grid=(grid,),
        in_specs=[
            pl.BlockSpec((_TC_CHUNK,), lambda i: (i,)),
            pl.BlockSpec((_TC_CHUNK,), lambda i: (i,)),
        ],
        out_specs=pl.BlockSpec(memory_space=pltpu.SMEM),
        out_shape=jax.ShapeDtypeStruct((2,), jnp.float32),
        scratch_shapes=[pltpu.VMEM((2, _TC_ROWS, 128), jnp.float32)],
    )(output, target)


def kernel(output, target):
    n = output.shape[0]
    n_sc = _SC_UNITS * _SC_UNIT
    n_tc = n - n_sc
    sc_parts = _make_sc_loss(n_tc, n_sc)(output, target)
    tc_parts = _tc_loss(output, target, n_tc)
    s = tc_parts[0] + jnp.sum(sc_parts[0])
    c = tc_parts[1] + jnp.sum(sc_parts[1])
    return s / c


# speedup vs baseline: 2.7082x; 1.0912x over previous
import functools

import jax
import jax.numpy as jnp
from jax import lax
from jax.experimental import pallas as pl
from jax.experimental.pallas import tpu as pltpu
from jax.experimental.pallas import tpu_sc as plsc

_SC_CH = 16384
_SC_UNIT = 524288
_SC_UNITS = 2

_TC_CHUNK = 524288
_TC_SUB = 16384
_TC_ROWS = _TC_SUB // 128

_LP = (3.51102136e-06, 0.999792362, -0.496977431, 0.314589174,
       -0.188780824, 0.0817256453, -0.0172077992)


def _log1p_poly(e):
    acc = jnp.full_like(e, _LP[6])
    for coef in _LP[5::-1]:
        acc = acc * e + coef
    return acc


def _masked_bce(x, t, log1p_fn):
    tf = t.astype(jnp.float32)
    zf = jnp.maximum(tf, 0.0)
    mf = jnp.minimum(tf + 1.0, 1.0)
    e = jnp.exp(-jnp.abs(x))
    sp = jnp.maximum(x, 0.0) + log1p_fn(e)
    return mf * sp - x * zf, mf


def _make_sc_loss(n_skip, n_sc):
    info = plsc.get_sparse_core_info()
    nc, ns = info.num_cores, info.num_subcores
    nw = nc * ns
    per_w = n_sc // nw
    n_ch = per_w // _SC_CH

    mesh = plsc.VectorSubcoreMesh(core_axis_name="c", subcore_axis_name="s")

    @functools.partial(
        pl.kernel, mesh=mesh,
        out_type=jax.ShapeDtypeStruct((2, nw, 16), jnp.float32),
        scratch_types=[
            pltpu.VMEM((_SC_CH,), jnp.float32),
            pltpu.VMEM((_SC_CH,), jnp.int32),
            pltpu.VMEM((2, 16), jnp.float32),
        ],
    )
    def sc_loss(x_hbm, t_hbm, out_hbm, x_v, t_v, part_v):
        wid = lax.axis_index("s") * nc + lax.axis_index("c")
        base = n_skip + wid * per_w

        def chunk_body(ci, carry):
            pltpu.sync_copy(x_hbm.at[pl.ds(base + ci * _SC_CH, _SC_CH)], x_v)
            pltpu.sync_copy(t_hbm.at[pl.ds(base + ci * _SC_CH, _SC_CH)], t_v)

            def vec_body(j, sc2):
                s2, c2 = sc2
                x = x_v[pl.ds(j * 16, 16)]
                t = t_v[pl.ds(j * 16, 16)]
                ds, dc = _masked_bce(x, t, _log1p_poly)
                return s2 + ds, c2 + dc

            return lax.fori_loop(0, _SC_CH // 16, vec_body, carry, unroll=8)

        z = jnp.zeros((16,), jnp.float32)
        s, c = lax.fori_loop(0, n_ch, chunk_body, (z, z))
        part_v[0] = s
        part_v[1] = c
        pltpu.sync_copy(part_v.at[0], out_hbm.at[0, wid])
        pltpu.sync_copy(part_v.at[1], out_hbm.at[1, wid])

    return sc_loss


def _tc_body(x_ref, t_ref, out_ref, acc_ref):
    i = pl.program_id(0)

    @pl.when(i == 0)
    def _init():
        acc_ref[...] = jnp.zeros_like(acc_ref)

    def step(j, carry):
        s, c = carry
        x = x_ref[pl.ds(j * _TC_SUB, _TC_SUB)].reshape(_TC_ROWS, 128)
        t = t_ref[pl.ds(j * _TC_SUB, _TC_SUB)].reshape(_TC_ROWS, 128)
        ds, dc = _masked_bce(x, t, jnp.log1p)
        return s + ds, c + dc

    init = (jnp.zeros((_TC_ROWS, 128), jnp.float32),
            jnp.zeros((_TC_ROWS, 128), jnp.float32))
    s, c = jax.lax.fori_loop(0, _TC_CHUNK // _TC_SUB, step, init, unroll=4)
    acc_ref[0] += s
    acc_ref[1] += c

    @pl.when(i == pl.num_programs(0) - 1)
    def _fin():
        out_ref[0] = jnp.sum(acc_ref[0])
        out_ref[1] = jnp.sum(acc_ref[1])


def _tc_loss(output, target, n_tc):
    grid = n_tc // _TC_CHUNK
    return pl.pallas_call(
        _tc_body,
        grid=(grid,),
        in_specs=[
            pl.BlockSpec((_TC_CHUNK,), lambda i: (i,)),
            pl.BlockSpec((_TC_CHUNK,), lambda i: (i,)),
        ],
        out_specs=pl.BlockSpec(memory_space=pltpu.SMEM),
        out_shape=jax.ShapeDtypeStruct((2,), jnp.float32),
        scratch_shapes=[pltpu.VMEM((2, _TC_ROWS, 128), jnp.float32)],
    )(output, target)


def kernel(output, target):
    n = output.shape[0]
    n_sc = _SC_UNITS * _SC_UNIT
    n_tc = n - n_sc
    sc_parts = _make_sc_loss(n_tc, n_sc)(output, target)
    tc_parts = _tc_loss(output, target, n_tc)
    s = tc_parts[0] + jnp.sum(sc_parts[0])
    c = tc_parts[1] + jnp.sum(sc_parts[1])
    return s / c
